# Initial kernel scaffold; baseline (speedup 1.0000x reference)
import jax
import jax.numpy as jnp
from jax.experimental import pallas as pl


def kernel(x, Wg, bias_g, W1, W2, W3, W1s, W2s, W3s):
    raise NotImplementedError("write your pallas kernel here")


# speedup vs baseline: 1.1380x; 1.1380x over previous
import functools

import jax
import jax.numpy as jnp
from jax.experimental import pallas as pl
from jax.experimental.pallas import tpu as pltpu

DIM = 1024
INTER = 512
E = 8
TOKENS = 2048
TILE = 256


def _moe_kernel(x_ref, wg_ref, bias_ref, w1_ref, w2_ref, w3_ref,
                w1s_ref, w2s_ref, w3s_ref, out_ref):
    xf = x_ref[...]
    xb = xf.astype(jnp.bfloat16)

    scores = jax.lax.dot_general(
        xb, wg_ref[...].astype(jnp.bfloat16), (((1,), (1,)), ((), ())),
        preferred_element_type=jnp.float32)
    s = jnp.sqrt(jax.nn.softplus(scores))
    sb = s + bias_ref[0:1, :]

    eidx = jax.lax.broadcasted_iota(jnp.int32, (TILE, E), 1)
    neg_inf = jnp.float32(-jnp.inf)

    m1 = jnp.max(sb, axis=1, keepdims=True)
    a1 = jnp.min(jnp.where(sb == m1, eidx, E), axis=1, keepdims=True)
    sb2 = jnp.where(eidx == a1, neg_inf, sb)
    m2 = jnp.max(sb2, axis=1, keepdims=True)
    a2 = jnp.min(jnp.where(sb2 == m2, eidx, E), axis=1, keepdims=True)

    oh1 = (eidx == a1).astype(jnp.float32)
    oh2 = (eidx == a2).astype(jnp.float32)
    w_1 = jnp.sum(s * oh1, axis=1, keepdims=True)
    w_2 = jnp.sum(s * oh2, axis=1, keepdims=True)
    denom = w_1 + w_2
    combine = (w_1 * oh1 + w_2 * oh2) / denom

    dn_in = (((1,), (1,)), ((), ()))

    def ffn(w1e, w3e, w2e, scale):
        g = jax.lax.dot_general(xb, w1e, dn_in,
                                preferred_element_type=jnp.float32)
        u = jax.lax.dot_general(xb, w3e, dn_in,
                                preferred_element_type=jnp.float32)
        h = (g * jax.nn.sigmoid(g)) * u * scale
        return jax.lax.dot_general(h.astype(jnp.bfloat16), w2e, dn_in,
                                   preferred_element_type=jnp.float32)

    acc = ffn(w1s_ref[...], w3s_ref[...], w2s_ref[...], 1.0)
    for e in range(E):
        acc += ffn(w1_ref[e], w3_ref[e], w2_ref[e], combine[:, e:e + 1])

    out_ref[...] = acc


@functools.partial(jax.jit, static_argnames=())
def kernel(x, Wg, bias_g, W1, W2, W3, W1s, W2s, W3s):
    shape = x.shape
    xf = x.reshape(-1, shape[-1]).astype(jnp.float32)
    t = xf.shape[0]
    bias2d = jnp.broadcast_to(bias_g.astype(jnp.float32), (8, E))

    bf = jnp.bfloat16
    grid = (t // TILE,)
    full = lambda a: pl.BlockSpec(a.shape, lambda i: (0,) * a.ndim)

    out = pl.pallas_call(
        _moe_kernel,
        grid=grid,
        in_specs=[
            pl.BlockSpec((TILE, DIM), lambda i: (i, 0)),
            full(Wg), full(bias2d),
            full(W1), full(W2), full(W3),
            full(W1s), full(W2s), full(W3s),
        ],
        out_specs=pl.BlockSpec((TILE, DIM), lambda i: (i, 0)),
        out_shape=jax.ShapeDtypeStruct((t, DIM), jnp.float32),
        compiler_params=pltpu.CompilerParams(
            dimension_semantics=("parallel",),
        ),
    )(xf, Wg.astype(jnp.float32), bias2d,
      W1.astype(bf), W2.astype(bf), W3.astype(bf),
      W1s.astype(bf), W2s.astype(bf), W3s.astype(bf))
    return out.reshape(shape)
